# baseline (speedup 1.0000x reference)
import jax
import jax.numpy as jnp
from jax.experimental import pallas as pl

BLOCK_S = 256


def _add_pe_kernel(x_ref, pe_ref, o_ref):
    o_ref[...] = x_ref[...] + pe_ref[...]


def kernel(x, pe):
    batch, seq_len, d_model = x.shape
    grid = (seq_len // BLOCK_S,)
    return pl.pallas_call(
        _add_pe_kernel,
        grid=grid,
        in_specs=[
            pl.BlockSpec((batch, BLOCK_S, d_model), lambda i: (0, i, 0)),
            pl.BlockSpec((BLOCK_S, d_model), lambda i: (i, 0)),
        ],
        out_specs=pl.BlockSpec((batch, BLOCK_S, d_model), lambda i: (0, i, 0)),
        out_shape=jax.ShapeDtypeStruct(x.shape, x.dtype),
    )(x, pe)


# speedup vs baseline: 1.0095x; 1.0095x over previous
import jax
import jax.numpy as jnp
from jax.experimental import pallas as pl

BLOCK_S = 2048


def _add_pe_kernel(x_ref, pe_ref, o_ref):
    o_ref[...] = x_ref[...] + pe_ref[...]


def kernel(x, pe):
    batch, seq_len, d_model = x.shape
    xf = x.reshape(batch * seq_len, d_model)
    n_s = seq_len // BLOCK_S
    out = pl.pallas_call(
        _add_pe_kernel,
        grid=(n_s, batch),
        in_specs=[
            pl.BlockSpec((BLOCK_S, d_model), lambda i, b: (b * n_s + i, 0)),
            pl.BlockSpec((BLOCK_S, d_model), lambda i, b: (i, 0)),
        ],
        out_specs=pl.BlockSpec((BLOCK_S, d_model), lambda i, b: (b * n_s + i, 0)),
        out_shape=jax.ShapeDtypeStruct(xf.shape, x.dtype),
    )(xf, pe)
    return out.reshape(batch, seq_len, d_model)
